# baseline (device time: 11300 ns/iter reference)
import jax
import jax.numpy as jnp
from jax import lax
from jax.experimental import pallas as pl
from jax.experimental.pallas import tpu as pltpu

BM = 512


def kernel(x):
    m, n = x.shape
    nblk = m // BM
    half = nblk // 2

    def body(x_ref, out_ref, acc_ref, comm_ref, send_sems, recv_sems):
        i = pl.program_id(0)
        my_x = lax.axis_index("x")
        my_y = lax.axis_index("y")
        nbr = (1 - my_x, my_y)

        barrier_sem = pltpu.get_barrier_semaphore()

        @pl.when(i == 0)
        def _():
            pl.semaphore_signal(
                barrier_sem, inc=1, device_id=nbr,
                device_id_type=pl.DeviceIdType.MESH,
            )
            acc_ref[...] = jnp.zeros_like(acc_ref)

        acc_ref[...] += jnp.sum(
            x_ref[...].reshape(BM // 8, 8, n), axis=0
        )

        def exchange(slot):
            return pltpu.make_async_remote_copy(
                src_ref=comm_ref.at[2 * slot],
                dst_ref=comm_ref.at[2 * slot + 1],
                send_sem=send_sems.at[slot],
                recv_sem=recv_sems.at[slot],
                device_id=nbr,
                device_id_type=pl.DeviceIdType.MESH,
            )

        @pl.when(i == half - 1)
        def _():
            comm_ref[0] = jnp.sum(acc_ref[...], axis=0, keepdims=True)
            pl.semaphore_wait(barrier_sem, 1)
            exchange(0).start()
            acc_ref[...] = jnp.zeros_like(acc_ref)

        @pl.when(i == nblk - 1)
        def _():
            comm_ref[2] = jnp.sum(acc_ref[...], axis=0, keepdims=True)
            rdma2 = exchange(1)
            rdma2.start()
            rdma1 = exchange(0)
            rdma1.wait()
            partial = comm_ref[0] + comm_ref[1] + comm_ref[2]
            rdma2.wait()
            out_ref[...] = partial + comm_ref[3]

    return pl.pallas_call(
        body,
        grid=(nblk,),
        out_shape=jax.ShapeDtypeStruct((1, n), jnp.float32),
        in_specs=[
            pl.BlockSpec((BM, n), lambda i: (i, 0), memory_space=pltpu.VMEM)
        ],
        out_specs=pl.BlockSpec((1, n), lambda i: (0, 0), memory_space=pltpu.VMEM),
        scratch_shapes=[
            pltpu.VMEM((8, n), jnp.float32),
            pltpu.VMEM((4, 1, n), jnp.float32),
            pltpu.SemaphoreType.DMA((2,)),
            pltpu.SemaphoreType.DMA((2,)),
        ],
        compiler_params=pltpu.CompilerParams(collective_id=0),
    )(x)
